# baseline (speedup 1.0000x reference)
import functools

import jax
import jax.numpy as jnp
from jax import lax
from jax.experimental import pallas as pl
from jax.experimental.pallas import tpu as pltpu
from jax.experimental.pallas import tpu_sc as plsc

D_IN = 768
D_MODEL = 1152
NUM_POS = 332
BATCH = 4096


_NC = 2
_NS = 16
_NW = _NC * _NS
_B_PER_W = BATCH // _NW
_CHUNK = 64
_N_CHUNKS = _B_PER_W // _CHUNK


@functools.cache
def _make_sc_gather():
    mesh = plsc.VectorSubcoreMesh(core_axis_name="c", subcore_axis_name="s")

    @functools.partial(
        pl.kernel,
        out_type=jax.ShapeDtypeStruct((BATCH, D_MODEL), jnp.float32),
        mesh=mesh,
        scratch_types=[
            pltpu.VMEM((_CHUNK,), jnp.int32),
            pltpu.VMEM((_CHUNK, D_MODEL), jnp.float32),
            pltpu.SemaphoreType.DMA,
        ],
    )
    def _sc_gather(table_hbm, idx_hbm, out_hbm, idx_v, rows_v, sem):
        wid = lax.axis_index("s") * _NC + lax.axis_index("c")
        base = wid * _B_PER_W
        for c in range(_N_CHUNKS):
            start = base + c * _CHUNK
            pltpu.sync_copy(idx_hbm.at[pl.ds(start, _CHUNK)], idx_v)
            pltpu.async_copy(table_hbm.at[idx_v], rows_v, sem).wait()
            pltpu.sync_copy(rows_v, out_hbm.at[pl.ds(start, _CHUNK)])

    return _sc_gather


_BM = 512


_BN = D_MODEL // 3


def _tc_body(phrase_ref, w1_ref, b1_ref, wmu_ref, bmu_ref, wvar_ref,
             bvar_ref, mean_ref, var_ref, h_ref):
    @pl.when(pl.program_id(1) == 0)
    def _():
        h = jnp.dot(phrase_ref[...], w1_ref[...],
                    preferred_element_type=jnp.float32)
        h_ref[...] = jnp.maximum(h + b1_ref[...], 0.0)

    h = h_ref[...]
    mean_ref[...] = jnp.dot(h, wmu_ref[...],
                            preferred_element_type=jnp.float32) + bmu_ref[...]
    logvar = jnp.dot(h, wvar_ref[...],
                     preferred_element_type=jnp.float32) + bvar_ref[...]
    var_ref[...] = jnp.exp(logvar)


def _tc_encoder(phrase, W1, b1, Wmu, bmu, Wvar, bvar):
    grid = (BATCH // _BM, D_MODEL // _BN)
    out_shape = jax.ShapeDtypeStruct((BATCH, D_MODEL), jnp.float32)
    return pl.pallas_call(
        _tc_body,
        grid=grid,
        in_specs=[
            pl.BlockSpec((_BM, D_IN), lambda i, j: (i, 0)),
            pl.BlockSpec((D_IN, D_MODEL), lambda i, j: (0, 0)),
            pl.BlockSpec((1, D_MODEL), lambda i, j: (0, 0)),
            pl.BlockSpec((D_MODEL, _BN), lambda i, j: (0, j)),
            pl.BlockSpec((1, _BN), lambda i, j: (0, j)),
            pl.BlockSpec((D_MODEL, _BN), lambda i, j: (0, j)),
            pl.BlockSpec((1, _BN), lambda i, j: (0, j)),
        ],
        out_specs=[
            pl.BlockSpec((_BM, _BN), lambda i, j: (i, j)),
            pl.BlockSpec((_BM, _BN), lambda i, j: (i, j)),
        ],
        out_shape=[out_shape, out_shape],
        scratch_shapes=[pltpu.VMEM((_BM, D_MODEL), jnp.float32)],
        compiler_params=pltpu.CompilerParams(
            dimension_semantics=("arbitrary", "arbitrary"),
        ),
    )(phrase, W1, b1, Wmu, bmu, Wvar, bvar)


def _add_body(mean_ref, pos_ref, feat_ref):
    feat_ref[...] = mean_ref[...] + pos_ref[...]


def _tc_add(mean, pos_emb):
    n_blocks = BATCH // _BM
    spec = pl.BlockSpec((_BM, D_MODEL), lambda i: (i, 0))
    return pl.pallas_call(
        _add_body,
        grid=(n_blocks,),
        in_specs=[spec, spec],
        out_specs=spec,
        out_shape=jax.ShapeDtypeStruct((BATCH, D_MODEL), jnp.float32),
        compiler_params=pltpu.CompilerParams(
            dimension_semantics=("arbitrary",),
        ),
    )(mean, pos_emb)


def kernel(phrase, position, W1, b1, Wmu, bmu, Wvar, bvar, pos_table):
    pos_emb = _make_sc_gather()(pos_table, position.astype(jnp.int32))
    mean, var = _tc_encoder(
        phrase, W1, b1.reshape(1, D_MODEL), Wmu, bmu.reshape(1, D_MODEL),
        Wvar, bvar.reshape(1, D_MODEL))
    feature = _tc_add(mean, pos_emb)
    return (feature, mean, var)


# speedup vs baseline: 1.3431x; 1.3431x over previous
import functools

import jax
import jax.numpy as jnp
from jax import lax
from jax.experimental import pallas as pl
from jax.experimental.pallas import tpu as pltpu
from jax.experimental.pallas import tpu_sc as plsc

D_IN = 768
D_MODEL = 1152
NUM_POS = 332
BATCH = 4096


_NC = 2
_NS = 16
_NW = _NC * _NS
_B_PER_W = BATCH // _NW
_CHUNK = 32
_N_CHUNKS = _B_PER_W // _CHUNK

@functools.cache
def _make_sc_gather():
    mesh = plsc.VectorSubcoreMesh(core_axis_name="c", subcore_axis_name="s")

    @functools.partial(
        pl.kernel,
        out_type=jax.ShapeDtypeStruct((BATCH, D_MODEL), jnp.float32),
        mesh=mesh,
        scratch_types=[
            pltpu.VMEM((_B_PER_W,), jnp.int32),
            pltpu.VMEM((2 * _CHUNK, D_MODEL), jnp.float32),
            pltpu.SemaphoreType.DMA,
            pltpu.SemaphoreType.DMA,
            pltpu.SemaphoreType.DMA,
        ],
    )
    def _sc_gather(table_hbm, idx_hbm, out_hbm, idx_v, rows_v, gsem,
                   wsem0, wsem1):
        wid = lax.axis_index("s") * _NC + lax.axis_index("c")
        base = wid * _B_PER_W
        wsems = (wsem0, wsem1)
        pltpu.sync_copy(idx_hbm.at[pl.ds(base, _B_PER_W)], idx_v)
        writes = [None, None]
        for c in range(_N_CHUNKS):
            slot = c % 2
            buf = rows_v.at[pl.ds(slot * _CHUNK, _CHUNK)]
            if writes[slot] is not None:
                writes[slot].wait()
            pltpu.async_copy(
                table_hbm.at[idx_v.at[pl.ds(c * _CHUNK, _CHUNK)]], buf,
                gsem).wait()
            writes[slot] = pltpu.async_copy(
                buf, out_hbm.at[pl.ds(base + c * _CHUNK, _CHUNK)],
                wsems[slot])
        for w in writes:
            w.wait()

    return _sc_gather


_BM = 512


def _tc_body(phrase_ref, pos_ref, w1_ref, b1_ref, wmu_ref, bmu_ref,
             wvar_ref, bvar_ref, feat_ref, mean_ref, var_ref):
    h = jnp.dot(phrase_ref[...], w1_ref[...],
                preferred_element_type=jnp.float32)
    h = jnp.maximum(h + b1_ref[...], 0.0)
    mean = jnp.dot(h, wmu_ref[...],
                   preferred_element_type=jnp.float32) + bmu_ref[...]
    logvar = jnp.dot(h, wvar_ref[...],
                     preferred_element_type=jnp.float32) + bvar_ref[...]
    mean_ref[...] = mean
    var_ref[...] = jnp.exp(logvar)
    feat_ref[...] = mean + pos_ref[...]


def _tc_encoder(phrase, pos_emb, W1, b1, Wmu, bmu, Wvar, bvar):
    n_blocks = BATCH // _BM
    row_spec = pl.BlockSpec((_BM, D_IN), lambda i: (i, 0))
    row_out = pl.BlockSpec((_BM, D_MODEL), lambda i: (i, 0))
    full = lambda shape: pl.BlockSpec(shape, lambda i: (0, 0))
    out_shape = jax.ShapeDtypeStruct((BATCH, D_MODEL), jnp.float32)
    return pl.pallas_call(
        _tc_body,
        grid=(n_blocks,),
        in_specs=[
            row_spec,
            row_out,
            full((D_IN, D_MODEL)),
            full((1, D_MODEL)),
            full((D_MODEL, D_MODEL)),
            full((1, D_MODEL)),
            full((D_MODEL, D_MODEL)),
            full((1, D_MODEL)),
        ],
        out_specs=[row_out, row_out, row_out],
        out_shape=[out_shape, out_shape, out_shape],
        compiler_params=pltpu.CompilerParams(
            dimension_semantics=("arbitrary",),
        ),
    )(phrase, pos_emb, W1, b1, Wmu, bmu, Wvar, bvar)


def kernel(phrase, position, W1, b1, Wmu, bmu, Wvar, bvar, pos_table):
    pos_emb = _make_sc_gather()(pos_table, position.astype(jnp.int32))
    feature, mean, var = _tc_encoder(
        phrase, pos_emb, W1,
        b1.reshape(1, D_MODEL), Wmu, bmu.reshape(1, D_MODEL),
        Wvar, bvar.reshape(1, D_MODEL))
    return (feature, mean, var)


# speedup vs baseline: 1.3597x; 1.0124x over previous
import functools

import jax
import jax.numpy as jnp
from jax import lax
from jax.experimental import pallas as pl
from jax.experimental.pallas import tpu as pltpu
from jax.experimental.pallas import tpu_sc as plsc

D_IN = 768
D_MODEL = 1152
NUM_POS = 332
BATCH = 4096


_NC = 2
_NS = 16
_NW = _NC * _NS
_B_PER_W = BATCH // _NW
_CHUNK = 64
_N_CHUNKS = _B_PER_W // _CHUNK

@functools.cache
def _make_sc_gather():
    mesh = plsc.VectorSubcoreMesh(core_axis_name="c", subcore_axis_name="s")

    @functools.partial(
        pl.kernel,
        out_type=jax.ShapeDtypeStruct((BATCH, D_MODEL), jnp.float32),
        mesh=mesh,
        scratch_types=[
            pltpu.VMEM((_B_PER_W,), jnp.int32),
            pltpu.VMEM((_CHUNK, D_MODEL), jnp.float32),
            pltpu.SemaphoreType.DMA,
        ],
    )
    def _sc_gather(table_hbm, idx_hbm, out_hbm, idx_v, rows_v, sem):
        wid = lax.axis_index("s") * _NC + lax.axis_index("c")
        base = wid * _B_PER_W
        pltpu.sync_copy(idx_hbm.at[pl.ds(base, _B_PER_W)], idx_v)
        for c in range(_N_CHUNKS):
            pltpu.async_copy(
                table_hbm.at[idx_v.at[pl.ds(c * _CHUNK, _CHUNK)]], rows_v,
                sem).wait()
            pltpu.sync_copy(
                rows_v, out_hbm.at[pl.ds(base + c * _CHUNK, _CHUNK)])

    return _sc_gather


_BM = 512


def _tc_body(phrase_ref, pos_ref, w1_ref, b1_ref, wmu_ref, bmu_ref,
             wvar_ref, bvar_ref, feat_ref, mean_ref, var_ref):
    h = jnp.dot(phrase_ref[...], w1_ref[...],
                preferred_element_type=jnp.float32)
    h = jnp.maximum(h + b1_ref[...], 0.0)
    mean = jnp.dot(h, wmu_ref[...],
                   preferred_element_type=jnp.float32) + bmu_ref[...]
    logvar = jnp.dot(h, wvar_ref[...],
                     preferred_element_type=jnp.float32) + bvar_ref[...]
    mean_ref[...] = mean
    var_ref[...] = jnp.exp(logvar)
    feat_ref[...] = mean + pos_ref[...]


def _tc_encoder(phrase, pos_emb, W1, b1, Wmu, bmu, Wvar, bvar):
    n_blocks = BATCH // _BM
    row_spec = pl.BlockSpec((_BM, D_IN), lambda i: (i, 0))
    row_out = pl.BlockSpec((_BM, D_MODEL), lambda i: (i, 0))
    full = lambda shape: pl.BlockSpec(shape, lambda i: (0, 0))
    out_shape = jax.ShapeDtypeStruct((BATCH, D_MODEL), jnp.float32)
    return pl.pallas_call(
        _tc_body,
        grid=(n_blocks,),
        in_specs=[
            row_spec,
            row_out,
            full((D_IN, D_MODEL)),
            full((1, D_MODEL)),
            full((D_MODEL, D_MODEL)),
            full((1, D_MODEL)),
            full((D_MODEL, D_MODEL)),
            full((1, D_MODEL)),
        ],
        out_specs=[row_out, row_out, row_out],
        out_shape=[out_shape, out_shape, out_shape],
        compiler_params=pltpu.CompilerParams(
            dimension_semantics=("parallel",),
        ),
    )(phrase, pos_emb, W1, b1, Wmu, bmu, Wvar, bvar)


def kernel(phrase, position, W1, b1, Wmu, bmu, Wvar, bvar, pos_table):
    pos_emb = _make_sc_gather()(pos_table, position.astype(jnp.int32))
    feature, mean, var = _tc_encoder(
        phrase, pos_emb, W1,
        b1.reshape(1, D_MODEL), Wmu, bmu.reshape(1, D_MODEL),
        Wvar, bvar.reshape(1, D_MODEL))
    return (feature, mean, var)
